# Initial kernel scaffold; baseline (speedup 1.0000x reference)
import jax
import jax.numpy as jnp
from jax.experimental import pallas as pl


def kernel(x):
    raise NotImplementedError("write your pallas kernel here")


# speedup vs baseline: 4.8857x; 4.8857x over previous
import jax
import jax.numpy as jnp
from jax.experimental import pallas as pl

K = 64
ROWS_PER_BLOCK = 8
N_COLS = 32768


def _topk_mask_block(x_ref, o_ref):
    x = x_ref[...]
    b = jax.lax.bitcast_convert_type(x, jnp.int32)
    keys = jnp.where(b < 0, b ^ jnp.int32(0x7FFFFFFF), b)

    def body(i, t):
        bit = 31 - i
        cand = t + jnp.left_shift(jnp.int32(1), bit)
        cnt = jnp.sum((keys >= cand).astype(jnp.int32), axis=1, keepdims=True)
        return jnp.where(cnt >= K, cand, t)

    t0 = jnp.full((x.shape[0], 1), jnp.int32(-2147483648))
    t = jax.lax.fori_loop(0, 32, body, t0)
    o_ref[...] = jnp.where(keys >= t, x, 0.0)


def kernel(x):
    m, n = x.shape
    grid = (m // ROWS_PER_BLOCK,)
    return pl.pallas_call(
        _topk_mask_block,
        grid=grid,
        in_specs=[pl.BlockSpec((ROWS_PER_BLOCK, n), lambda i: (i, 0))],
        out_specs=pl.BlockSpec((ROWS_PER_BLOCK, n), lambda i: (i, 0)),
        out_shape=jax.ShapeDtypeStruct((m, n), x.dtype),
    )(x)


# speedup vs baseline: 8.4166x; 1.7227x over previous
import jax
import jax.numpy as jnp
from jax.experimental import pallas as pl

K = 64
ROWS_PER_BLOCK = 8
N_COLS = 32768


def _topk_mask_block(x_ref, o_ref):
    x = x_ref[...]
    b = jax.lax.bitcast_convert_type(x, jnp.int32)
    keys = jnp.where(b < 0, b ^ jnp.int32(0x7FFFFFFF), b)

    n = keys.shape[1]
    nsplit = 8
    w = n // nsplit

    def body(i, t):
        bit = 31 - i
        cand = t + jnp.left_shift(jnp.int32(1), bit)
        cnt = sum(
            jnp.sum(
                (keys[:, j * w:(j + 1) * w] >= cand).astype(jnp.int32),
                axis=1,
                keepdims=True,
            )
            for j in range(nsplit)
        )
        return jnp.where(cnt >= K, cand, t)

    t0 = jnp.full((x.shape[0], 1), jnp.int32(-2147483648))
    t = jax.lax.fori_loop(0, 32, body, t0)
    o_ref[...] = jnp.where(keys >= t, x, 0.0)


def kernel(x):
    m, n = x.shape
    grid = (m // ROWS_PER_BLOCK,)
    return pl.pallas_call(
        _topk_mask_block,
        grid=grid,
        in_specs=[pl.BlockSpec((ROWS_PER_BLOCK, n), lambda i: (i, 0))],
        out_specs=pl.BlockSpec((ROWS_PER_BLOCK, n), lambda i: (i, 0)),
        out_shape=jax.ShapeDtypeStruct((m, n), x.dtype),
    )(x)


# speedup vs baseline: 12.3288x; 1.4648x over previous
import jax
import jax.numpy as jnp
from jax.experimental import pallas as pl

K = 64
ROWS_PER_BLOCK = 32
NSPLIT = 8


def _topk_mask_block(x_ref, o_ref):
    x = x_ref[...]
    b = jax.lax.bitcast_convert_type(x, jnp.int32)
    keys = jnp.where(b < 0, b ^ jnp.int32(0x7FFFFFFF), b)

    n = keys.shape[1]
    w = n // NSPLIT

    def body(i, t):
        bit = 31 - i
        cand = t + jnp.left_shift(jnp.int32(1), bit)
        cnt = sum(
            jnp.sum(
                (keys[:, j * w:(j + 1) * w] >= cand).astype(jnp.int32),
                axis=1,
                keepdims=True,
            )
            for j in range(NSPLIT)
        )
        return jnp.where(cnt >= K, cand, t)

    t0 = jnp.full((x.shape[0], 1), jnp.int32(-2147483648))
    t = jax.lax.fori_loop(0, 32, body, t0)
    o_ref[...] = jnp.where(keys >= t, x, 0.0)


def kernel(x):
    m, n = x.shape
    grid = (m // ROWS_PER_BLOCK,)
    return pl.pallas_call(
        _topk_mask_block,
        grid=grid,
        in_specs=[pl.BlockSpec((ROWS_PER_BLOCK, n), lambda i: (i, 0))],
        out_specs=pl.BlockSpec((ROWS_PER_BLOCK, n), lambda i: (i, 0)),
        out_shape=jax.ShapeDtypeStruct((m, n), x.dtype),
    )(x)


# speedup vs baseline: 13.3107x; 1.0796x over previous
import jax
import jax.numpy as jnp
from jax.experimental import pallas as pl

K = 64
ROWS_PER_BLOCK = 64
NSPLIT = 8


def _topk_mask_block(x_ref, o_ref):
    x = x_ref[...]
    b = jax.lax.bitcast_convert_type(x, jnp.int32)
    keys = jnp.where(b < 0, b ^ jnp.int32(0x7FFFFFFF), b)

    n = keys.shape[1]
    w = n // NSPLIT

    def body(i, t):
        bit = 31 - i
        cand = t + jnp.left_shift(jnp.int32(1), bit)
        cnt = sum(
            jnp.sum(
                (keys[:, j * w:(j + 1) * w] >= cand).astype(jnp.int32),
                axis=1,
                keepdims=True,
            )
            for j in range(NSPLIT)
        )
        return jnp.where(cnt >= K, cand, t)

    t0 = jnp.full((x.shape[0], 1), jnp.int32(-2147483648))
    t = jax.lax.fori_loop(0, 32, body, t0)
    o_ref[...] = jnp.where(keys >= t, x, 0.0)


def kernel(x):
    m, n = x.shape
    grid = (m // ROWS_PER_BLOCK,)
    return pl.pallas_call(
        _topk_mask_block,
        grid=grid,
        in_specs=[pl.BlockSpec((ROWS_PER_BLOCK, n), lambda i: (i, 0))],
        out_specs=pl.BlockSpec((ROWS_PER_BLOCK, n), lambda i: (i, 0)),
        out_shape=jax.ShapeDtypeStruct((m, n), x.dtype),
    )(x)
